# baseline (speedup 1.0000x reference)
import functools
import jax
import jax.numpy as jnp
from jax import lax
from jax.experimental import pallas as pl
from jax.experimental.pallas import tpu as pltpu
from jax.experimental.pallas import tpu_sc as plsc

N = 16384
V = 1000

_info = plsc.get_sparse_core_info()
NC, NS, L = _info.num_cores, _info.num_subcores, _info.num_lanes
NW = NC * NS
ROWS_PER_W = N // NW
CHUNK = 16
NCHUNK = ROWS_PER_W // CHUNK
NB = 4

_mesh = plsc.VectorSubcoreMesh(core_axis_name="c", subcore_axis_name="s")


@functools.partial(
    pl.kernel,
    mesh=_mesh,
    out_type=jax.ShapeDtypeStruct((N, V), jnp.float32),
    scratch_types=[
        pltpu.VMEM((ROWS_PER_W,), jnp.int32),
        pltpu.VMEM((CHUNK, V), jnp.float32),
        pltpu.VMEM((CHUNK, V), jnp.float32),
        pltpu.VMEM((CHUNK, V), jnp.float32),
        pltpu.VMEM((CHUNK, V), jnp.float32),
        pltpu.SemaphoreType.DMA,
        pltpu.SemaphoreType.DMA,
        pltpu.SemaphoreType.DMA,
        pltpu.SemaphoreType.DMA,
    ],
    compiler_params=pltpu.CompilerParams(needs_layout_passes=False),
)
def _sc_onehot(x_hbm, out_hbm, idx_v, buf0, buf1, buf2, buf3,
               sem0, sem1, sem2, sem3):
    wid = lax.axis_index("s") * NC + lax.axis_index("c")
    base_row = wid * ROWS_PER_W

    pltpu.sync_copy(x_hbm.at[pl.ds(base_row, ROWS_PER_W)], idx_v)

    bufs = (buf0, buf1, buf2, buf3)
    sems = (sem0, sem1, sem2, sem3)
    zeros16 = jnp.zeros((L,), jnp.float32)
    ones16 = jnp.ones((L,), jnp.float32)
    lane = lax.iota(jnp.int32, L)

    def _zero_buf(buf):
        def _zero_row(r, carry):
            def _zero_k(k, c2):
                off = pl.multiple_of(k * L, L)
                buf[r, pl.ds(off, L)] = zeros16
                return c2

            lax.fori_loop(0, V // L, _zero_k, 0)
            rows = jnp.full((L,), r, jnp.int32)
            plsc.store_scatter(buf, [rows, (V - L) + lane], zeros16)
            return carry

        lax.fori_loop(0, CHUNK, _zero_row, 0)

    copies = [None] * NCHUNK
    positions = [None] * NCHUNK
    for c in range(NCHUNK):
        b = c % NB
        if c < NB:
            _zero_buf(bufs[b])
        else:
            copies[c - NB].wait()
            rows, cols = positions[c - NB]
            plsc.store_scatter(bufs[b], [rows, cols], zeros16)
        xs = idx_v[pl.ds(c * CHUNK, L)]
        plsc.store_scatter(bufs[b], [lane, xs], ones16)
        positions[c] = (lane, xs)
        row0 = base_row + c * CHUNK
        cp = pltpu.make_async_copy(
            bufs[b],
            out_hbm.at[pl.ds(row0, CHUNK), :],
            sems[b],
        )
        cp.start()
        copies[c] = cp
    for d in range(NB):
        copies[NCHUNK - NB + d].wait()


def kernel(X):
    return _sc_onehot(X)


# speedup vs baseline: 1.0139x; 1.0139x over previous
import functools
import jax
import jax.numpy as jnp
from jax import lax
from jax.experimental import pallas as pl
from jax.experimental.pallas import tpu as pltpu
from jax.experimental.pallas import tpu_sc as plsc

N = 16384
V = 1000

_info = plsc.get_sparse_core_info()
NC, NS, L = _info.num_cores, _info.num_subcores, _info.num_lanes
NW = NC * NS
ROWS_PER_W = N // NW
CHUNK = 16
NCHUNK = ROWS_PER_W // CHUNK
NB = 4

_mesh = plsc.VectorSubcoreMesh(core_axis_name="c", subcore_axis_name="s")


@functools.partial(
    pl.kernel,
    mesh=_mesh,
    out_type=jax.ShapeDtypeStruct((N, V), jnp.float32),
    scratch_types=[
        pltpu.VMEM((ROWS_PER_W,), jnp.int32),
        pltpu.VMEM((CHUNK, V), jnp.float32),
        pltpu.VMEM((CHUNK, V), jnp.float32),
        pltpu.VMEM((CHUNK, V), jnp.float32),
        pltpu.VMEM((CHUNK, V), jnp.float32),
        pltpu.SemaphoreType.DMA,
        pltpu.SemaphoreType.DMA,
        pltpu.SemaphoreType.DMA,
        pltpu.SemaphoreType.DMA,
    ],
    compiler_params=pltpu.CompilerParams(needs_layout_passes=False),
)
def _sc_onehot(x_hbm, out_hbm, idx_v, buf0, buf1, buf2, buf3,
               sem0, sem1, sem2, sem3):
    wid = lax.axis_index("s") * NC + lax.axis_index("c")
    base_row = wid * ROWS_PER_W

    pltpu.sync_copy(x_hbm.at[pl.ds(base_row, ROWS_PER_W)], idx_v)

    bufs = (buf0, buf1, buf2, buf3)
    sems = (sem0, sem1, sem2, sem3)
    zeros16 = jnp.zeros((L,), jnp.float32)
    ones16 = jnp.ones((L,), jnp.float32)
    lane = lax.iota(jnp.int32, L)

    def _zero_buf(buf):
        def _zero_row(r, carry):
            def _zero_k(k, c2):
                off = pl.multiple_of(k * L, L)
                buf[r, pl.ds(off, L)] = zeros16
                return c2

            lax.fori_loop(0, V // L, _zero_k, 0)
            rows = jnp.full((L,), r, jnp.int32)
            plsc.store_scatter(buf, [rows, (V - L) + lane], zeros16)
            return carry

        lax.fori_loop(0, CHUNK, _zero_row, 0)

    for buf in bufs:
        _zero_buf(buf)

    copies = [None] * NCHUNK
    positions = [None] * NCHUNK
    for c in range(NCHUNK):
        b = c % NB
        if c >= NB:
            copies[c - NB].wait()
            rows, cols = positions[c - NB]
            plsc.store_scatter(bufs[b], [rows, cols], zeros16)
        xs = idx_v[pl.ds(c * CHUNK, L)]
        plsc.store_scatter(bufs[b], [lane, xs], ones16)
        positions[c] = (lane, xs)
        row0 = base_row + c * CHUNK
        cp = pltpu.make_async_copy(
            bufs[b],
            out_hbm.at[pl.ds(row0, CHUNK), :],
            sems[b],
        )
        cp.start()
        copies[c] = cp
    for d in range(NB):
        copies[NCHUNK - NB + d].wait()


def kernel(X):
    return _sc_onehot(X)


# speedup vs baseline: 1.1303x; 1.1148x over previous
import functools
import jax
import jax.numpy as jnp
from jax import lax
from jax.experimental import pallas as pl
from jax.experimental.pallas import tpu as pltpu
from jax.experimental.pallas import tpu_sc as plsc

N = 16384
V = 1000

_info = plsc.get_sparse_core_info()
NC, NS, L = _info.num_cores, _info.num_subcores, _info.num_lanes
NW = NC * NS
ROWS_PER_W = N // NW
CHUNK = 16
NCHUNK = ROWS_PER_W // CHUNK
NB = 4

_mesh = plsc.VectorSubcoreMesh(core_axis_name="c", subcore_axis_name="s")


@functools.partial(
    pl.kernel,
    mesh=_mesh,
    out_type=jax.ShapeDtypeStruct((N, V), jnp.float32),
    scratch_types=[
        pltpu.VMEM((ROWS_PER_W,), jnp.int32),
        pltpu.VMEM((CHUNK, V), jnp.float32),
        pltpu.VMEM((CHUNK, V), jnp.float32),
        pltpu.VMEM((CHUNK, V), jnp.float32),
        pltpu.VMEM((CHUNK, V), jnp.float32),
        pltpu.SemaphoreType.DMA,
        pltpu.SemaphoreType.DMA,
        pltpu.SemaphoreType.DMA,
        pltpu.SemaphoreType.DMA,
    ],
    compiler_params=pltpu.CompilerParams(needs_layout_passes=False),
)
def _sc_onehot(x_hbm, out_hbm, idx_v, buf0, buf1, buf2, buf3,
               sem0, sem1, sem2, sem3):
    wid = lax.axis_index("s") * NC + lax.axis_index("c")
    base_row = wid * ROWS_PER_W

    pltpu.sync_copy(x_hbm.at[pl.ds(base_row, ROWS_PER_W)], idx_v)

    bufs = (buf0, buf1, buf2, buf3)
    sems = (sem0, sem1, sem2, sem3)
    zeros16 = jnp.zeros((L,), jnp.float32)
    ones16 = jnp.ones((L,), jnp.float32)
    lane = lax.iota(jnp.int32, L)

    def _zero_row(r, carry):
        def _zero_k(k, c2):
            off = pl.multiple_of(k * L, L)
            for buf in bufs:
                buf[r, pl.ds(off, L)] = zeros16
            return c2

        lax.fori_loop(0, V // L, _zero_k, 0)
        rows = jnp.full((L,), r, jnp.int32)
        for buf in bufs:
            plsc.store_scatter(buf, [rows, (V - L) + lane], zeros16)
        return carry

    lax.fori_loop(0, CHUNK, _zero_row, 0)

    copies = [None] * NCHUNK
    positions = [None] * NCHUNK
    for c in range(NCHUNK):
        b = c % NB
        if c >= NB:
            copies[c - NB].wait()
            rows, cols = positions[c - NB]
            plsc.store_scatter(bufs[b], [rows, cols], zeros16)
        xs = idx_v[pl.ds(c * CHUNK, L)]
        plsc.store_scatter(bufs[b], [lane, xs], ones16)
        positions[c] = (lane, xs)
        row0 = base_row + c * CHUNK
        cp = pltpu.make_async_copy(
            bufs[b],
            out_hbm.at[pl.ds(row0, CHUNK), :],
            sems[b],
        )
        cp.start()
        copies[c] = cp
    for d in range(NB):
        copies[NCHUNK - NB + d].wait()


def kernel(X):
    return _sc_onehot(X)
